# baseline (speedup 1.0000x reference)
import functools

import jax
import jax.numpy as jnp
from jax import lax
from jax.experimental import pallas as pl
from jax.experimental.pallas import tpu as pltpu
from jax.experimental.pallas import tpu_sc as plsc

N = 10000
E = 320000
NP = 10240
NC = 2
NS = 16
NW = NC * NS
EPT = E // NW
RPT = NP // NS

DK = 80
DCHUNK = EPT // DK
DNB = 5

K = 40
NCHUNK = EPT // K
NBUF = 5


def _sc_mesh():
    return plsc.VectorSubcoreMesh(core_axis_name="c", subcore_axis_name="s",
                                  num_cores=NC, num_subcores=NS)


def _sc_degree(dst, ones_k, zeros_npf):
    @functools.partial(
        pl.kernel,
        out_type=jax.ShapeDtypeStruct((NC * NP, 128), jnp.float32),
        mesh=_sc_mesh(),
        scratch_types=[
            pltpu.VMEM((EPT,), jnp.int32),
            pltpu.VMEM((DK, 128), jnp.float32),
            pltpu.VMEM_SHARED((NP, 128), jnp.float32),
            pltpu.SemaphoreType.DMA,
        ],
    )
    def deg_kernel(dst_hbm, ones_hbm, zeros_hbm, out_hbm, idx_all, ones_v,
                   acc, sem_s):
        c = lax.axis_index("c")
        s = lax.axis_index("s")
        row0 = s * RPT
        pltpu.sync_copy(zeros_hbm.at[pl.ds(row0, RPT)], acc.at[pl.ds(row0, RPT)])
        pltpu.sync_copy(ones_hbm, ones_v)
        wid = c * NS + s
        pltpu.sync_copy(dst_hbm.at[pl.ds(wid * EPT, EPT)], idx_all)
        plsc.subcore_barrier()

        def dchunk(j):
            return idx_all.at[pl.ds(j * DK, DK)]

        def body(m, carry):
            @pl.when(m > 0)
            def _():
                for t in range(DNB):
                    pltpu.make_async_copy(
                        ones_v, acc.at[dchunk((m - 1) * DNB + t)], sem_s
                    ).wait()
            for t in range(DNB):
                pltpu.async_copy(ones_v, acc.at[dchunk(m * DNB + t)],
                                 sem_s, add=True)
            return carry

        lax.fori_loop(0, DCHUNK // DNB, body, 0)
        for t in range(DNB):
            pltpu.make_async_copy(
                ones_v, acc.at[dchunk(DCHUNK - DNB + t)], sem_s).wait()
        plsc.subcore_barrier()
        pltpu.sync_copy(acc.at[pl.ds(row0, RPT)],
                        out_hbm.at[pl.ds(c * NP + row0, RPT)])

    return deg_kernel(dst, ones_k, zeros_npf)


def _sc_segsum(table, src, dst, zeros_npf):
    @functools.partial(
        pl.kernel,
        out_type=jax.ShapeDtypeStruct((NC * NP, 128), jnp.float32),
        mesh=_sc_mesh(),
        scratch_types=[
            pltpu.VMEM((EPT,), jnp.int32),
            pltpu.VMEM((EPT,), jnp.int32),
            pltpu.VMEM((NBUF, K, 128), jnp.float32),
            pltpu.VMEM_SHARED((NP, 128), jnp.float32),
            pltpu.SemaphoreType.DMA,
            pltpu.SemaphoreType.DMA,
            pltpu.SemaphoreType.DMA,
            pltpu.SemaphoreType.DMA,
            pltpu.SemaphoreType.DMA,
            pltpu.SemaphoreType.DMA,
            pltpu.SemaphoreType.DMA,
            pltpu.SemaphoreType.DMA,
            pltpu.SemaphoreType.DMA,
            pltpu.SemaphoreType.DMA,
        ],
    )
    def seg_kernel(table_hbm, src_hbm, dst_hbm, zeros_hbm, out_hbm,
                   idx_s, idx_d, rows, acc,
                   sg0, sg1, sg2, sg3, sg4, ss0, ss1, ss2, ss3, ss4):
        c = lax.axis_index("c")
        s = lax.axis_index("s")
        row0 = s * RPT
        pltpu.sync_copy(zeros_hbm.at[pl.ds(row0, RPT)], acc.at[pl.ds(row0, RPT)])
        wid = c * NS + s
        pltpu.sync_copy(src_hbm.at[pl.ds(wid * EPT, EPT)], idx_s)
        pltpu.sync_copy(dst_hbm.at[pl.ds(wid * EPT, EPT)], idx_d)
        plsc.subcore_barrier()
        sgs = [sg0, sg1, sg2, sg3, sg4]
        sss = [ss0, ss1, ss2, ss3, ss4]

        def g_start(r, j):
            pltpu.async_copy(
                table_hbm.at[idx_s.at[pl.ds(j * K, K)]],
                rows.at[r], sgs[r])

        def g_wait(r, j):
            pltpu.make_async_copy(
                table_hbm.at[idx_s.at[pl.ds(j * K, K)]],
                rows.at[r], sgs[r]).wait()

        def s_start(r, j):
            pltpu.async_copy(
                rows.at[r], acc.at[idx_d.at[pl.ds(j * K, K)]],
                sss[r], add=True)

        def s_wait(r, j):
            pltpu.make_async_copy(
                rows.at[r], acc.at[idx_d.at[pl.ds(j * K, K)]],
                sss[r]).wait()

        for r in range(NBUF):
            g_start(r, r)

        def body(m, carry):
            for r in range(NBUF):
                j = NBUF * m + r
                g_wait(r, j)
                s_start(r, j)
            for r in range(NBUF):
                j = NBUF * m + r
                s_wait(r, j)
                g_start(r, j + NBUF)
            return carry

        lax.fori_loop(0, NCHUNK // NBUF - 1, body, 0)
        for r in range(NBUF):
            j = NCHUNK - NBUF + r
            g_wait(r, j)
            s_start(r, j)
            s_wait(r, j)
        plsc.subcore_barrier()
        pltpu.sync_copy(acc.at[pl.ds(row0, RPT)],
                        out_hbm.at[pl.ds(c * NP + row0, RPT)])

    return seg_kernel(table, src, dst, zeros_npf)


def _tc_xw(x_p, W1):
    def body(x_ref, w_ref, xw_ref):
        xw_ref[...] = jnp.dot(x_ref[...], w_ref[...],
                              preferred_element_type=jnp.float32)

    return pl.pallas_call(
        body,
        out_shape=jax.ShapeDtypeStruct((NP, 128), jnp.float32),
    )(x_p, W1)


def _tc_prep(xw, deg_parts):
    def body(xw_ref, deg_ref, t_ref, dis_ref):
        d = deg_ref[0][:, 0:1] + deg_ref[1][:, 0:1] + 1.0
        dis = lax.rsqrt(d)
        t_ref[...] = xw_ref[...] * dis
        dis_ref[...] = dis

    return pl.pallas_call(
        body,
        out_shape=(
            jax.ShapeDtypeStruct((NP, 128), jnp.float32),
            jax.ShapeDtypeStruct((NP, 1), jnp.float32),
        ),
    )(xw, deg_parts)


def _tc_mid(S_parts, dis, T_prev, b_row, Wn):
    def body(s_ref, dis_ref, t_ref, b_ref, w_ref, tn_ref):
        dis = dis_ref[...]
        agg = (s_ref[0] + s_ref[1] + t_ref[...]) * dis + b_ref[...]
        h = jnp.maximum(agg, 0.0)
        xw = jnp.dot(h, w_ref[...], preferred_element_type=jnp.float32)
        tn_ref[...] = xw * dis

    return pl.pallas_call(
        body,
        out_shape=jax.ShapeDtypeStruct((NP, 128), jnp.float32),
    )(S_parts, dis, T_prev, b_row, Wn)


def _tc_final(S_parts, dis, T_prev, b_row):
    def body(s_ref, dis_ref, t_ref, b_ref, out_ref):
        agg = ((s_ref[0] + s_ref[1] + t_ref[...]) * dis_ref[...]
               + b_ref[...])
        out_ref[...] = agg[:N, :16]

    return pl.pallas_call(
        body,
        out_shape=jax.ShapeDtypeStruct((N, 16), jnp.float32),
    )(S_parts, dis, T_prev, b_row)


def _pad_cols(a, width=128):
    return jnp.pad(a, [(0, 0)] * (a.ndim - 1) + [(0, width - a.shape[-1])])


def kernel(x, edge_index, W1, b1, W2, b2, W3, b3, W4, b4):
    src = edge_index[0]
    dst = edge_index[1]
    x_p = jnp.pad(x, ((0, NP - N), (0, 0)))
    ones_k = jnp.ones((DK, 128), jnp.float32)
    zeros_npf = jnp.zeros((NP, 128), jnp.float32)

    W2p = jnp.pad(W2, ((0, 0), (0, 64)))
    W3p = jnp.pad(W3, ((0, 64), (0, 96)))
    W4p = jnp.pad(W4, ((0, 96), (0, 112)))

    xw1 = _tc_xw(x_p, W1)
    deg_flat = _sc_degree(dst, ones_k, zeros_npf)
    deg_parts = deg_flat.reshape(NC, NP, 128)

    T, dis = _tc_prep(xw1, deg_parts)

    for b, Wn in ((b1, W2p), (b2, W3p), (b3, W4p)):
        S_flat = _sc_segsum(T, src, dst, zeros_npf)
        S_parts = S_flat.reshape(NC, NP, 128)
        T = _tc_mid(S_parts, dis, T, _pad_cols(b.reshape(1, -1)), Wn)

    S_flat = _sc_segsum(T, src, dst, zeros_npf)
    S_parts = S_flat.reshape(NC, NP, 128)
    return _tc_final(S_parts, dis, T, _pad_cols(b4.reshape(1, -1)))


# speedup vs baseline: 1.0120x; 1.0120x over previous
import functools

import jax
import jax.numpy as jnp
from jax import lax
from jax.experimental import pallas as pl
from jax.experimental.pallas import tpu as pltpu
from jax.experimental.pallas import tpu_sc as plsc

N = 10000
E = 320000
NP = 10240
NC = 2
NS = 16
NW = NC * NS
EPT = E // NW
RPT = NP // NS

DK = 80
DCHUNK = EPT // DK
DNB = 5

K = 40
NCHUNK = EPT // K
NBUF = 5


def _sc_mesh():
    return plsc.VectorSubcoreMesh(core_axis_name="c", subcore_axis_name="s",
                                  num_cores=NC, num_subcores=NS)


def _sc_degree(dst, ones_k, zeros_npf):
    @functools.partial(
        pl.kernel,
        out_type=jax.ShapeDtypeStruct((NC * NP, 128), jnp.float32),
        mesh=_sc_mesh(),
        scratch_types=[
            pltpu.VMEM((EPT,), jnp.int32),
            pltpu.VMEM((DK, 128), jnp.float32),
            pltpu.VMEM_SHARED((NP, 128), jnp.float32),
            pltpu.SemaphoreType.DMA,
        ],
    )
    def deg_kernel(dst_hbm, ones_hbm, zeros_hbm, out_hbm, idx_all, ones_v,
                   acc, sem_s):
        c = lax.axis_index("c")
        s = lax.axis_index("s")
        row0 = s * RPT
        wid = c * NS + s
        dz = pltpu.async_copy(zeros_hbm.at[pl.ds(row0, RPT)],
                              acc.at[pl.ds(row0, RPT)], sem_s)
        d1 = pltpu.async_copy(ones_hbm, ones_v, sem_s)
        d2 = pltpu.async_copy(dst_hbm.at[pl.ds(wid * EPT, EPT)], idx_all, sem_s)
        dz.wait()
        d1.wait()
        d2.wait()
        plsc.subcore_barrier()

        def dchunk(j):
            return idx_all.at[pl.ds(j * DK, DK)]

        def body(m, carry):
            @pl.when(m > 0)
            def _():
                for t in range(DNB):
                    pltpu.make_async_copy(
                        ones_v, acc.at[dchunk((m - 1) * DNB + t)], sem_s
                    ).wait()
            for t in range(DNB):
                pltpu.async_copy(ones_v, acc.at[dchunk(m * DNB + t)],
                                 sem_s, add=True)
            return carry

        lax.fori_loop(0, DCHUNK // DNB, body, 0)
        for t in range(DNB):
            pltpu.make_async_copy(
                ones_v, acc.at[dchunk(DCHUNK - DNB + t)], sem_s).wait()
        plsc.subcore_barrier()
        pltpu.sync_copy(acc.at[pl.ds(row0, RPT)],
                        out_hbm.at[pl.ds(c * NP + row0, RPT)])

    return deg_kernel(dst, ones_k, zeros_npf)


def _sc_segsum(table, src, dst, zeros_npf):
    @functools.partial(
        pl.kernel,
        out_type=jax.ShapeDtypeStruct((NC * NP, 128), jnp.float32),
        mesh=_sc_mesh(),
        scratch_types=[
            pltpu.VMEM((EPT,), jnp.int32),
            pltpu.VMEM((EPT,), jnp.int32),
            pltpu.VMEM((NBUF, K, 128), jnp.float32),
            pltpu.VMEM_SHARED((NP, 128), jnp.float32),
            pltpu.SemaphoreType.DMA,
            pltpu.SemaphoreType.DMA,
            pltpu.SemaphoreType.DMA,
            pltpu.SemaphoreType.DMA,
            pltpu.SemaphoreType.DMA,
            pltpu.SemaphoreType.DMA,
            pltpu.SemaphoreType.DMA,
            pltpu.SemaphoreType.DMA,
            pltpu.SemaphoreType.DMA,
            pltpu.SemaphoreType.DMA,
        ],
    )
    def seg_kernel(table_hbm, src_hbm, dst_hbm, zeros_hbm, out_hbm,
                   idx_s, idx_d, rows, acc,
                   sg0, sg1, sg2, sg3, sg4, ss0, ss1, ss2, ss3, ss4):
        c = lax.axis_index("c")
        s = lax.axis_index("s")
        row0 = s * RPT
        wid = c * NS + s
        dz = pltpu.async_copy(zeros_hbm.at[pl.ds(row0, RPT)],
                              acc.at[pl.ds(row0, RPT)], sg0)
        d1 = pltpu.async_copy(src_hbm.at[pl.ds(wid * EPT, EPT)], idx_s, sg1)
        d2 = pltpu.async_copy(dst_hbm.at[pl.ds(wid * EPT, EPT)], idx_d, sg2)
        dz.wait()
        d1.wait()
        d2.wait()
        plsc.subcore_barrier()
        sgs = [sg0, sg1, sg2, sg3, sg4]
        sss = [ss0, ss1, ss2, ss3, ss4]

        def g_start(r, j):
            pltpu.async_copy(
                table_hbm.at[idx_s.at[pl.ds(j * K, K)]],
                rows.at[r], sgs[r])

        def g_wait(r, j):
            pltpu.make_async_copy(
                table_hbm.at[idx_s.at[pl.ds(j * K, K)]],
                rows.at[r], sgs[r]).wait()

        def s_start(r, j):
            pltpu.async_copy(
                rows.at[r], acc.at[idx_d.at[pl.ds(j * K, K)]],
                sss[r], add=True)

        def s_wait(r, j):
            pltpu.make_async_copy(
                rows.at[r], acc.at[idx_d.at[pl.ds(j * K, K)]],
                sss[r]).wait()

        for r in range(NBUF):
            g_start(r, r)

        def body(m, carry):
            for r in range(NBUF):
                j = NBUF * m + r
                g_wait(r, j)
                s_start(r, j)
            for r in range(NBUF):
                j = NBUF * m + r
                s_wait(r, j)
                g_start(r, j + NBUF)
            return carry

        lax.fori_loop(0, NCHUNK // NBUF - 1, body, 0)
        for r in range(NBUF):
            j = NCHUNK - NBUF + r
            g_wait(r, j)
            s_start(r, j)
        for r in range(NBUF):
            s_wait(r, NCHUNK - NBUF + r)
        plsc.subcore_barrier()
        pltpu.sync_copy(acc.at[pl.ds(row0, RPT)],
                        out_hbm.at[pl.ds(c * NP + row0, RPT)])

    return seg_kernel(table, src, dst, zeros_npf)


def _tc_xw(x_p, W1):
    def body(x_ref, w_ref, xw_ref):
        xw_ref[...] = jnp.dot(x_ref[...], w_ref[...],
                              preferred_element_type=jnp.float32)

    return pl.pallas_call(
        body,
        out_shape=jax.ShapeDtypeStruct((NP, 128), jnp.float32),
    )(x_p, W1)


def _tc_prep(xw, deg_parts):
    def body(xw_ref, deg_ref, t_ref, dis_ref):
        d = deg_ref[0][:, 0:1] + deg_ref[1][:, 0:1] + 1.0
        dis = lax.rsqrt(d)
        t_ref[...] = xw_ref[...] * dis
        dis_ref[...] = dis

    return pl.pallas_call(
        body,
        out_shape=(
            jax.ShapeDtypeStruct((NP, 128), jnp.float32),
            jax.ShapeDtypeStruct((NP, 1), jnp.float32),
        ),
    )(xw, deg_parts)


def _tc_mid(S_parts, dis, T_prev, b_row, Wn):
    def body(s_ref, dis_ref, t_ref, b_ref, w_ref, tn_ref):
        dis = dis_ref[...]
        agg = (s_ref[0] + s_ref[1] + t_ref[...]) * dis + b_ref[...]
        h = jnp.maximum(agg, 0.0)
        xw = jnp.dot(h, w_ref[...], preferred_element_type=jnp.float32)
        tn_ref[...] = xw * dis

    return pl.pallas_call(
        body,
        out_shape=jax.ShapeDtypeStruct((NP, 128), jnp.float32),
    )(S_parts, dis, T_prev, b_row, Wn)


def _tc_final(S_parts, dis, T_prev, b_row):
    def body(s_ref, dis_ref, t_ref, b_ref, out_ref):
        agg = ((s_ref[0] + s_ref[1] + t_ref[...]) * dis_ref[...]
               + b_ref[...])
        out_ref[...] = agg[:N, :16]

    return pl.pallas_call(
        body,
        out_shape=jax.ShapeDtypeStruct((N, 16), jnp.float32),
    )(S_parts, dis, T_prev, b_row)


def _pad_cols(a, width=128):
    return jnp.pad(a, [(0, 0)] * (a.ndim - 1) + [(0, width - a.shape[-1])])


def kernel(x, edge_index, W1, b1, W2, b2, W3, b3, W4, b4):
    src = edge_index[0]
    dst = edge_index[1]
    x_p = jnp.pad(x, ((0, NP - N), (0, 0)))
    ones_k = jnp.ones((DK, 128), jnp.float32)
    zeros_npf = jnp.zeros((NP, 128), jnp.float32)

    W2p = jnp.pad(W2, ((0, 0), (0, 64)))
    W3p = jnp.pad(W3, ((0, 64), (0, 96)))
    W4p = jnp.pad(W4, ((0, 96), (0, 112)))

    xw1 = _tc_xw(x_p, W1)
    deg_flat = _sc_degree(dst, ones_k, zeros_npf)
    deg_parts = deg_flat.reshape(NC, NP, 128)

    T, dis = _tc_prep(xw1, deg_parts)

    for b, Wn in ((b1, W2p), (b2, W3p), (b3, W4p)):
        S_flat = _sc_segsum(T, src, dst, zeros_npf)
        S_parts = S_flat.reshape(NC, NP, 128)
        T = _tc_mid(S_parts, dis, T, _pad_cols(b.reshape(1, -1)), Wn)

    S_flat = _sc_segsum(T, src, dst, zeros_npf)
    S_parts = S_flat.reshape(NC, NP, 128)
    return _tc_final(S_parts, dis, T, _pad_cols(b4.reshape(1, -1)))


# speedup vs baseline: 1.0140x; 1.0020x over previous
import functools

import jax
import jax.numpy as jnp
from jax import lax
from jax.experimental import pallas as pl
from jax.experimental.pallas import tpu as pltpu
from jax.experimental.pallas import tpu_sc as plsc

N = 10000
E = 320000
NP = 10240
NC = 2
NS = 16
NW = NC * NS
EPT = E // NW
RPT = NP // NS

DK = 80
DCHUNK = EPT // DK
DNB = 5

K = 40
NCHUNK = EPT // K
NBUF = 5


def _sc_mesh():
    return plsc.VectorSubcoreMesh(core_axis_name="c", subcore_axis_name="s",
                                  num_cores=NC, num_subcores=NS)


def _sc_degree(dst, ones_k, zeros_npf):
    @functools.partial(
        pl.kernel,
        out_type=jax.ShapeDtypeStruct((NC * NP, 128), jnp.float32),
        mesh=_sc_mesh(),
        scratch_types=[
            pltpu.VMEM((EPT,), jnp.int32),
            pltpu.VMEM((DK, 128), jnp.float32),
            pltpu.VMEM_SHARED((NP, 128), jnp.float32),
            pltpu.SemaphoreType.DMA,
        ],
    )
    def deg_kernel(dst_hbm, ones_hbm, zeros_hbm, out_hbm, idx_all, ones_v,
                   acc, sem_s):
        c = lax.axis_index("c")
        s = lax.axis_index("s")
        row0 = s * RPT
        wid = c * NS + s
        dz = pltpu.async_copy(zeros_hbm.at[pl.ds(row0, RPT)],
                              acc.at[pl.ds(row0, RPT)], sem_s)
        d1 = pltpu.async_copy(ones_hbm, ones_v, sem_s)
        d2 = pltpu.async_copy(dst_hbm.at[pl.ds(wid * EPT, EPT)], idx_all, sem_s)
        dz.wait()
        d1.wait()
        d2.wait()
        plsc.subcore_barrier()

        def dchunk(j):
            return idx_all.at[pl.ds(j * DK, DK)]

        def body(m, carry):
            @pl.when(m > 1)
            def _():
                for t in range(DNB):
                    pltpu.make_async_copy(
                        ones_v, acc.at[dchunk((m - 2) * DNB + t)], sem_s
                    ).wait()
            for t in range(DNB):
                pltpu.async_copy(ones_v, acc.at[dchunk(m * DNB + t)],
                                 sem_s, add=True)
            return carry

        lax.fori_loop(0, DCHUNK // DNB, body, 0)
        for t in range(2 * DNB):
            pltpu.make_async_copy(
                ones_v, acc.at[dchunk(DCHUNK - 2 * DNB + t)], sem_s).wait()
        plsc.subcore_barrier()
        pltpu.sync_copy(acc.at[pl.ds(row0, RPT)],
                        out_hbm.at[pl.ds(c * NP + row0, RPT)])

    return deg_kernel(dst, ones_k, zeros_npf)


def _sc_segsum(table, src, dst, zeros_npf):
    @functools.partial(
        pl.kernel,
        out_type=jax.ShapeDtypeStruct((NC * NP, 128), jnp.float32),
        mesh=_sc_mesh(),
        scratch_types=[
            pltpu.VMEM((EPT,), jnp.int32),
            pltpu.VMEM((EPT,), jnp.int32),
            pltpu.VMEM((NBUF, K, 128), jnp.float32),
            pltpu.VMEM_SHARED((NP, 128), jnp.float32),
            pltpu.SemaphoreType.DMA,
            pltpu.SemaphoreType.DMA,
            pltpu.SemaphoreType.DMA,
            pltpu.SemaphoreType.DMA,
            pltpu.SemaphoreType.DMA,
            pltpu.SemaphoreType.DMA,
            pltpu.SemaphoreType.DMA,
            pltpu.SemaphoreType.DMA,
            pltpu.SemaphoreType.DMA,
            pltpu.SemaphoreType.DMA,
        ],
    )
    def seg_kernel(table_hbm, src_hbm, dst_hbm, zeros_hbm, out_hbm,
                   idx_s, idx_d, rows, acc,
                   sg0, sg1, sg2, sg3, sg4, ss0, ss1, ss2, ss3, ss4):
        c = lax.axis_index("c")
        s = lax.axis_index("s")
        row0 = s * RPT
        wid = c * NS + s
        dz = pltpu.async_copy(zeros_hbm.at[pl.ds(row0, RPT)],
                              acc.at[pl.ds(row0, RPT)], sg0)
        d1 = pltpu.async_copy(src_hbm.at[pl.ds(wid * EPT, EPT)], idx_s, sg1)
        d2 = pltpu.async_copy(dst_hbm.at[pl.ds(wid * EPT, EPT)], idx_d, sg2)
        dz.wait()
        d1.wait()
        d2.wait()
        plsc.subcore_barrier()
        sgs = [sg0, sg1, sg2, sg3, sg4]
        sss = [ss0, ss1, ss2, ss3, ss4]

        def g_start(r, j):
            pltpu.async_copy(
                table_hbm.at[idx_s.at[pl.ds(j * K, K)]],
                rows.at[r], sgs[r])

        def g_wait(r, j):
            pltpu.make_async_copy(
                table_hbm.at[idx_s.at[pl.ds(j * K, K)]],
                rows.at[r], sgs[r]).wait()

        def s_start(r, j):
            pltpu.async_copy(
                rows.at[r], acc.at[idx_d.at[pl.ds(j * K, K)]],
                sss[r], add=True)

        def s_wait(r, j):
            pltpu.make_async_copy(
                rows.at[r], acc.at[idx_d.at[pl.ds(j * K, K)]],
                sss[r]).wait()

        for r in range(NBUF):
            g_start(r, r)

        def body(m, carry):
            for r in range(NBUF):
                j = NBUF * m + r
                g_wait(r, j)
                s_start(r, j)
            for r in range(NBUF):
                j = NBUF * m + r
                s_wait(r, j)
                g_start(r, j + NBUF)
            return carry

        lax.fori_loop(0, NCHUNK // NBUF - 1, body, 0)
        for r in range(NBUF):
            j = NCHUNK - NBUF + r
            g_wait(r, j)
            s_start(r, j)
        for r in range(NBUF):
            s_wait(r, NCHUNK - NBUF + r)
        plsc.subcore_barrier()
        pltpu.sync_copy(acc.at[pl.ds(row0, RPT)],
                        out_hbm.at[pl.ds(c * NP + row0, RPT)])

    return seg_kernel(table, src, dst, zeros_npf)


def _tc_xw(x_p, W1):
    def body(x_ref, w_ref, xw_ref):
        xw_ref[...] = jnp.dot(x_ref[...], w_ref[...],
                              preferred_element_type=jnp.float32)

    return pl.pallas_call(
        body,
        out_shape=jax.ShapeDtypeStruct((NP, 128), jnp.float32),
    )(x_p, W1)


def _tc_prep(xw, deg_parts):
    def body(xw_ref, deg_ref, t_ref, dis_ref):
        d = deg_ref[0][:, 0:1] + deg_ref[1][:, 0:1] + 1.0
        dis = lax.rsqrt(d)
        t_ref[...] = xw_ref[...] * dis
        dis_ref[...] = dis

    return pl.pallas_call(
        body,
        out_shape=(
            jax.ShapeDtypeStruct((NP, 128), jnp.float32),
            jax.ShapeDtypeStruct((NP, 1), jnp.float32),
        ),
    )(xw, deg_parts)


def _tc_mid(S_parts, dis, T_prev, b_row, Wn):
    def body(s_ref, dis_ref, t_ref, b_ref, w_ref, tn_ref):
        dis = dis_ref[...]
        agg = (s_ref[0] + s_ref[1] + t_ref[...]) * dis + b_ref[...]
        h = jnp.maximum(agg, 0.0)
        xw = jnp.dot(h, w_ref[...], preferred_element_type=jnp.float32)
        tn_ref[...] = xw * dis

    return pl.pallas_call(
        body,
        out_shape=jax.ShapeDtypeStruct((NP, 128), jnp.float32),
    )(S_parts, dis, T_prev, b_row, Wn)


def _tc_final(S_parts, dis, T_prev, b_row):
    def body(s_ref, dis_ref, t_ref, b_ref, out_ref):
        agg = ((s_ref[0] + s_ref[1] + t_ref[...]) * dis_ref[...]
               + b_ref[...])
        out_ref[...] = agg[:N, :16]

    return pl.pallas_call(
        body,
        out_shape=jax.ShapeDtypeStruct((N, 16), jnp.float32),
    )(S_parts, dis, T_prev, b_row)


def _pad_cols(a, width=128):
    return jnp.pad(a, [(0, 0)] * (a.ndim - 1) + [(0, width - a.shape[-1])])


def kernel(x, edge_index, W1, b1, W2, b2, W3, b3, W4, b4):
    src = edge_index[0]
    dst = edge_index[1]
    x_p = jnp.pad(x, ((0, NP - N), (0, 0)))
    ones_k = jnp.ones((DK, 128), jnp.float32)
    zeros_npf = jnp.zeros((NP, 128), jnp.float32)

    W2p = jnp.pad(W2, ((0, 0), (0, 64)))
    W3p = jnp.pad(W3, ((0, 64), (0, 96)))
    W4p = jnp.pad(W4, ((0, 96), (0, 112)))

    xw1 = _tc_xw(x_p, W1)
    deg_flat = _sc_degree(dst, ones_k, zeros_npf)
    deg_parts = deg_flat.reshape(NC, NP, 128)

    T, dis = _tc_prep(xw1, deg_parts)

    for b, Wn in ((b1, W2p), (b2, W3p), (b3, W4p)):
        S_flat = _sc_segsum(T, src, dst, zeros_npf)
        S_parts = S_flat.reshape(NC, NP, 128)
        T = _tc_mid(S_parts, dis, T, _pad_cols(b.reshape(1, -1)), Wn)

    S_flat = _sc_segsum(T, src, dst, zeros_npf)
    S_parts = S_flat.reshape(NC, NP, 128)
    return _tc_final(S_parts, dis, T, _pad_cols(b4.reshape(1, -1)))
